# baseline (speedup 1.0000x reference)
import functools
import math

import jax
import jax.numpy as jnp
from jax import lax
from jax.experimental import pallas as pl
from jax.experimental.pallas import tpu as pltpu
from jax.experimental.pallas import tpu_sc as plsc

E = 8
TOP_K = 2
D = 2048
F = 8192
T = 4096
A = T * TOP_K

TILE_M = 256
TSUB = 4
SUPER_M = TILE_M * TSUB
MAX_SUPER = 16
MAX_TILES = MAX_SUPER * TSUB
N_PAD = MAX_SUPER * SUPER_M
TILE_F = 1024
TM_L = 512

NC = 2
NS = 16
NW = NC * NS
LANES = 16
TPW = T // NW
CH = 16
NCHUNK = TPW // CH


def _logits_body(x_ref, wg_ref, out_ref):
    out_ref[...] = lax.dot_general(
        x_ref[...], wg_ref[...], (((1,), (1,)), ((), ())),
        preferred_element_type=jnp.float32)


def _logits_call(x, w_gate):
    return pl.pallas_call(
        _logits_body,
        grid=(T // TM_L,),
        in_specs=[
            pl.BlockSpec((TM_L, D), lambda i: (i, 0)),
            pl.BlockSpec((E, D), lambda i: (0, 0)),
        ],
        out_specs=pl.BlockSpec((TM_L, E), lambda i: (i, 0)),
        out_shape=jax.ShapeDtypeStruct((T, E), jnp.float32),
    )(x, w_gate)


def _route_body(lg_ref, pa_ref, pb_ref, ga_ref, gb_ref, te_ref, lv_ref,
                ls_ref):
    lg = lg_ref[...]
    col = lax.broadcasted_iota(jnp.int32, (T, E), 1)
    m1 = jnp.max(lg, axis=1, keepdims=True)
    e1 = jnp.min(jnp.where(lg == m1, col, E), axis=1, keepdims=True)
    oh1 = col == e1
    lg2 = jnp.where(oh1, -jnp.inf, lg)
    m2 = jnp.max(lg2, axis=1, keepdims=True)
    e2 = jnp.min(jnp.where(lg2 == m2, col, E), axis=1, keepdims=True)
    oh2 = col == e2
    dexp = jnp.exp(m2 - m1)
    g1 = 1.0 / (1.0 + dexp)
    g2 = dexp * g1

    ohc = (oh1 | oh2).astype(jnp.float32)
    tri = lax.broadcasted_iota(jnp.int32, (512, 512), 0) > \
        lax.broadcasted_iota(jnp.int32, (512, 512), 1)
    trif = tri.astype(jnp.float32)
    run = jnp.zeros((1, E), jnp.float32)
    parts = []
    for r in range(T // 512):
        blk = lax.slice(ohc, (512 * r, 0), (512 * (r + 1), E))
        c_blk = lax.dot_general(trif, blk, (((1,), (0,)), ((), ())),
                                preferred_element_type=jnp.float32) + run
        parts.append(c_blk)
        run = run + jnp.sum(blk, axis=0, keepdims=True)
    cpre = jnp.concatenate(parts, axis=0)
    counts = run

    nsuper = jnp.floor((counts + (SUPER_M - 1)) / SUPER_M)
    pad_rows = nsuper * SUPER_M
    up = (lax.broadcasted_iota(jnp.int32, (E, E), 0) <
          lax.broadcasted_iota(jnp.int32, (E, E), 1)).astype(jnp.float32)
    pad_off = lax.dot_general(pad_rows, up, (((1,), (0,)), ((), ())),
                              preferred_element_type=jnp.float32)

    oh1f = oh1.astype(jnp.float32)
    oh2f = oh2.astype(jnp.float32)
    rank1 = jnp.sum(cpre * oh1f, axis=1, keepdims=True)
    rank2 = jnp.sum(cpre * oh2f, axis=1, keepdims=True)
    off1 = jnp.sum(pad_off * oh1f, axis=1, keepdims=True)
    off2 = jnp.sum(pad_off * oh2f, axis=1, keepdims=True)
    pa_ref[...] = (off1 + rank1).astype(jnp.int32)
    pb_ref[...] = (off2 + rank2).astype(jnp.int32)
    ga_ref[...] = g1
    gb_ref[...] = g2

    pad_off_i = pad_off.astype(jnp.int32)
    counts_i = counts.astype(jnp.int32)
    seg_end_i = (pad_off + pad_rows).astype(jnp.int32)
    ti = lax.broadcasted_iota(jnp.int32, (MAX_TILES, E), 0) * TILE_M
    te = jnp.sum((ti >= seg_end_i).astype(jnp.int32), axis=1, keepdims=True)
    te_ref[...] = jnp.minimum(te, E - 1)
    lv = (ti >= pad_off_i) & (ti < pad_off_i + counts_i)
    lv_ref[...] = jnp.sum(lv.astype(jnp.int32), axis=1, keepdims=True)
    si = lax.broadcasted_iota(jnp.int32, (MAX_SUPER, E), 0) * SUPER_M
    ls = (si >= pad_off_i) & (si < pad_off_i + counts_i)
    ls_ref[...] = jnp.sum(ls.astype(jnp.int32), axis=1, keepdims=True)


def _route_call(logits):
    outs = pl.pallas_call(
        _route_body,
        in_specs=[pl.BlockSpec((T, E), lambda: (0, 0))],
        out_specs=[
            pl.BlockSpec((T, 1), lambda: (0, 0)),
            pl.BlockSpec((T, 1), lambda: (0, 0)),
            pl.BlockSpec((T, 1), lambda: (0, 0)),
            pl.BlockSpec((T, 1), lambda: (0, 0)),
            pl.BlockSpec((MAX_TILES, 1), lambda: (0, 0)),
            pl.BlockSpec((MAX_TILES, 1), lambda: (0, 0)),
            pl.BlockSpec((MAX_SUPER, 1), lambda: (0, 0)),
        ],
        out_shape=[
            jax.ShapeDtypeStruct((T, 1), jnp.int32),
            jax.ShapeDtypeStruct((T, 1), jnp.int32),
            jax.ShapeDtypeStruct((T, 1), jnp.float32),
            jax.ShapeDtypeStruct((T, 1), jnp.float32),
            jax.ShapeDtypeStruct((MAX_TILES, 1), jnp.int32),
            jax.ShapeDtypeStruct((MAX_TILES, 1), jnp.int32),
            jax.ShapeDtypeStruct((MAX_SUPER, 1), jnp.int32),
        ],
    )(logits)
    pa, pb, ga, gb, te, lv, ls = outs
    return (pa.reshape(T), pb.reshape(T), ga.reshape(T), gb.reshape(T),
            te.reshape(MAX_TILES), lv.reshape(MAX_TILES),
            ls.reshape(MAX_SUPER))


def _sc_mesh():
    return plsc.VectorSubcoreMesh(
        core_axis_name="c", subcore_axis_name="s",
        num_cores=NC, num_subcores=NS)


def _dispatch_body(x_hbm, pa_hbm, pb_hbm, xs_hbm, xbuf, ia, ib, sem):
    wid = lax.axis_index("s") * NC + lax.axis_index("c")
    base = wid * TPW

    def step(c, carry):
        tb = base + c * CH
        pltpu.sync_copy(x_hbm.at[pl.ds(tb, CH)], xbuf)
        pltpu.sync_copy(pa_hbm.at[pl.ds(tb, CH)], ia)
        pltpu.sync_copy(pb_hbm.at[pl.ds(tb, CH)], ib)
        cpa = pltpu.async_copy(xbuf, xs_hbm.at[ia], sem)
        cpb = pltpu.async_copy(xbuf, xs_hbm.at[ib], sem)
        cpa.wait()
        cpb.wait()
        return carry

    lax.fori_loop(0, NCHUNK, step, 0)


def _dispatch_call(x, pa, pb):
    fn = functools.partial(
        pl.kernel,
        out_type=jax.ShapeDtypeStruct((N_PAD, D // 2), jnp.int32),
        mesh=_sc_mesh(),
        scratch_types=[
            pltpu.VMEM((CH, D // 2), jnp.int32),
            pltpu.VMEM((CH,), jnp.int32),
            pltpu.VMEM((CH,), jnp.int32),
            pltpu.SemaphoreType.DMA,
        ],
    )(_dispatch_body)
    return fn(x, pa, pb)


def _ffn_body(te_ref, lv_ref, ls_ref, xs_ref, w1_ref, w2_ref, out_ref):
    f = pl.program_id(1)
    ts = pl.program_id(2)
    t = pl.program_id(0) * TSUB + ts

    @pl.when(lv_ref[t] > 0)
    def _():
        x = xs_ref[pl.ds(ts * TILE_M, TILE_M), :]
        w1 = w1_ref[0]
        h = lax.dot_general(x, w1, (((1,), (1,)), ((), ())),
                            preferred_element_type=jnp.float32)
        h = h * 0.5 * (1.0 + lax.erf(h * (1.0 / math.sqrt(2.0))))
        w2 = w2_ref[0]
        contrib = lax.dot_general(h.astype(jnp.bfloat16), w2,
                                  (((1,), (1,)), ((), ())),
                                  preferred_element_type=jnp.float32)

        @pl.when(f == 0)
        def _():
            out_ref[pl.ds(ts * TILE_M, TILE_M), :] = contrib

        @pl.when(f > 0)
        def _():
            out_ref[pl.ds(ts * TILE_M, TILE_M), :] = (
                out_ref[pl.ds(ts * TILE_M, TILE_M), :] + contrib)


def _ffn_call(tile_expert, live, live_super, xs, w_fc, w_proj):
    grid_spec = pltpu.PrefetchScalarGridSpec(
        num_scalar_prefetch=3,
        grid=(MAX_SUPER, F // TILE_F, TSUB),
        in_specs=[
            pl.BlockSpec((SUPER_M, D),
                         lambda s, f, ts, te, lv, ls:
                         (jnp.where(ls[s] > 0, s, 0), 0)),
            pl.BlockSpec((1, TILE_F, D),
                         lambda s, f, ts, te, lv, ls:
                         (te[s * TSUB + ts], f, 0)),
            pl.BlockSpec((1, D, TILE_F),
                         lambda s, f, ts, te, lv, ls:
                         (te[s * TSUB + ts], 0, f)),
        ],
        out_specs=pl.BlockSpec((SUPER_M, D),
                               lambda s, f, ts, te, lv, ls: (s, 0)),
    )
    return pl.pallas_call(
        _ffn_body,
        grid_spec=grid_spec,
        out_shape=jax.ShapeDtypeStruct((N_PAD, D), jnp.float32),
    )(tile_expert, live, live_super, xs, w_fc, w_proj)


def _combine_body(rows_hbm, pa_hbm, pb_hbm, ga_hbm, gb_hbm, y_hbm,
                  bufa, bufb, bufo, ia, ib, ga, gb, sem):
    wid = lax.axis_index("s") * NC + lax.axis_index("c")
    base = wid * TPW

    def step(c, carry):
        tb = base + c * CH
        pltpu.sync_copy(pa_hbm.at[pl.ds(tb, CH)], ia)
        pltpu.sync_copy(pb_hbm.at[pl.ds(tb, CH)], ib)
        pltpu.sync_copy(ga_hbm.at[pl.ds(tb, CH)], ga)
        pltpu.sync_copy(gb_hbm.at[pl.ds(tb, CH)], gb)
        cpa = pltpu.async_copy(rows_hbm.at[ia], bufa, sem)
        cpb = pltpu.async_copy(rows_hbm.at[ib], bufb, sem)
        cpa.wait()
        cpb.wait()
        gveca = ga[...]
        gvecb = gb[...]
        for i in range(CH):
            gva = gveca[i]
            gvb = gvecb[i]

            def lane(j, c2):
                sl = pl.ds(pl.multiple_of(j * LANES, LANES), LANES)
                bufo[i, sl] = gva * bufa[i, sl] + gvb * bufb[i, sl]
                return c2

            lax.fori_loop(0, D // LANES, lane, 0)
        pltpu.sync_copy(bufo, y_hbm.at[pl.ds(tb, CH)])
        return carry

    lax.fori_loop(0, NCHUNK, step, 0)


def _combine_call(rows, pa, pb, ga, gb):
    fn = functools.partial(
        pl.kernel,
        out_type=jax.ShapeDtypeStruct((T, D), jnp.float32),
        mesh=_sc_mesh(),
        scratch_types=[
            pltpu.VMEM((CH, D), jnp.float32),
            pltpu.VMEM((CH, D), jnp.float32),
            pltpu.VMEM((CH, D), jnp.float32),
            pltpu.VMEM((CH,), jnp.int32),
            pltpu.VMEM((CH,), jnp.int32),
            pltpu.VMEM((CH,), jnp.float32),
            pltpu.VMEM((CH,), jnp.float32),
            pltpu.SemaphoreType.DMA,
        ],
    )(_combine_body)
    return fn(rows, pa, pb, ga, gb)


def kernel(hidden_states, w_gate, w_fc, w_proj):
    b, s, d = hidden_states.shape
    x = hidden_states.reshape(-1, d)
    logits = _logits_call(x, w_gate)
    pa, pb, ga, gb, tile_expert, live, live_super = _route_call(logits)
    x16 = x.astype(jnp.bfloat16)
    x_i32 = lax.bitcast_convert_type(x16.reshape(T, D // 2, 2), jnp.int32)
    xs_i32 = _dispatch_call(x_i32, pa, pb)
    xs = lax.bitcast_convert_type(xs_i32, jnp.bfloat16).reshape(N_PAD, D)
    rows = _ffn_call(tile_expert, live, live_super, xs,
                     w_fc.astype(jnp.bfloat16), w_proj.astype(jnp.bfloat16))
    y = _combine_call(rows, pa, pb, ga, gb)
    return y.reshape(b, s, d), logits


# speedup vs baseline: 1.2690x; 1.2690x over previous
import functools
import math

import jax
import jax.numpy as jnp
from jax import lax
from jax.experimental import pallas as pl
from jax.experimental.pallas import tpu as pltpu
from jax.experimental.pallas import tpu_sc as plsc

E = 8
TOP_K = 2
D = 2048
F = 8192
T = 4096
A = T * TOP_K

TILE_M = 256
TSUB = 4
SUPER_M = TILE_M * TSUB
MAX_SUPER = 16
MAX_TILES = MAX_SUPER * TSUB
N_PAD = MAX_SUPER * SUPER_M
TILE_F = 512
TM_L = 512

NC = 2
NS = 16
NW = NC * NS
LANES = 16
TPW = T // NW
CH = 16
NCHUNK = TPW // CH


def _logits_body(x_ref, wg_ref, out_ref):
    out_ref[...] = lax.dot_general(
        x_ref[...], wg_ref[...], (((1,), (1,)), ((), ())),
        preferred_element_type=jnp.float32)


def _logits_call(x, w_gate):
    return pl.pallas_call(
        _logits_body,
        grid=(T // TM_L,),
        in_specs=[
            pl.BlockSpec((TM_L, D), lambda i: (i, 0)),
            pl.BlockSpec((E, D), lambda i: (0, 0)),
        ],
        out_specs=pl.BlockSpec((TM_L, E), lambda i: (i, 0)),
        out_shape=jax.ShapeDtypeStruct((T, E), jnp.float32),
    )(x, w_gate)


def _route_body(lg_ref, pa_ref, pb_ref, ga_ref, gb_ref, te_ref, lv_ref,
                ls_ref):
    lg = lg_ref[...]
    col = lax.broadcasted_iota(jnp.int32, (T, E), 1)
    m1 = jnp.max(lg, axis=1, keepdims=True)
    e1 = jnp.min(jnp.where(lg == m1, col, E), axis=1, keepdims=True)
    oh1 = col == e1
    lg2 = jnp.where(oh1, -jnp.inf, lg)
    m2 = jnp.max(lg2, axis=1, keepdims=True)
    e2 = jnp.min(jnp.where(lg2 == m2, col, E), axis=1, keepdims=True)
    oh2 = col == e2
    dexp = jnp.exp(m2 - m1)
    g1 = 1.0 / (1.0 + dexp)
    g2 = dexp * g1

    ohc = (oh1 | oh2).astype(jnp.float32)
    tri = lax.broadcasted_iota(jnp.int32, (512, 512), 0) > \
        lax.broadcasted_iota(jnp.int32, (512, 512), 1)
    trif = tri.astype(jnp.float32)
    run = jnp.zeros((1, E), jnp.float32)
    parts = []
    for r in range(T // 512):
        blk = lax.slice(ohc, (512 * r, 0), (512 * (r + 1), E))
        c_blk = lax.dot_general(trif, blk, (((1,), (0,)), ((), ())),
                                preferred_element_type=jnp.float32) + run
        parts.append(c_blk)
        run = run + jnp.sum(blk, axis=0, keepdims=True)
    cpre = jnp.concatenate(parts, axis=0)
    counts = run

    nsuper = jnp.floor((counts + (SUPER_M - 1)) / SUPER_M)
    pad_rows = nsuper * SUPER_M
    up = (lax.broadcasted_iota(jnp.int32, (E, E), 0) <
          lax.broadcasted_iota(jnp.int32, (E, E), 1)).astype(jnp.float32)
    pad_off = lax.dot_general(pad_rows, up, (((1,), (0,)), ((), ())),
                              preferred_element_type=jnp.float32)

    oh1f = oh1.astype(jnp.float32)
    oh2f = oh2.astype(jnp.float32)
    rank1 = jnp.sum(cpre * oh1f, axis=1, keepdims=True)
    rank2 = jnp.sum(cpre * oh2f, axis=1, keepdims=True)
    off1 = jnp.sum(pad_off * oh1f, axis=1, keepdims=True)
    off2 = jnp.sum(pad_off * oh2f, axis=1, keepdims=True)
    pa_ref[...] = (off1 + rank1).astype(jnp.int32)
    pb_ref[...] = (off2 + rank2).astype(jnp.int32)
    ga_ref[...] = g1
    gb_ref[...] = g2

    pad_off_i = pad_off.astype(jnp.int32)
    counts_i = counts.astype(jnp.int32)
    seg_end_i = (pad_off + pad_rows).astype(jnp.int32)
    ti = lax.broadcasted_iota(jnp.int32, (MAX_TILES, E), 0) * TILE_M
    te = jnp.sum((ti >= seg_end_i).astype(jnp.int32), axis=1, keepdims=True)
    te_ref[...] = jnp.minimum(te, E - 1)
    lv = (ti >= pad_off_i) & (ti < pad_off_i + counts_i)
    lv_ref[...] = jnp.sum(lv.astype(jnp.int32), axis=1, keepdims=True)
    si = lax.broadcasted_iota(jnp.int32, (MAX_SUPER, E), 0) * SUPER_M
    ls = (si >= pad_off_i) & (si < pad_off_i + counts_i)
    ls_ref[...] = jnp.sum(ls.astype(jnp.int32), axis=1, keepdims=True)


def _route_call(logits):
    outs = pl.pallas_call(
        _route_body,
        in_specs=[pl.BlockSpec((T, E), lambda: (0, 0))],
        out_specs=[
            pl.BlockSpec((T, 1), lambda: (0, 0)),
            pl.BlockSpec((T, 1), lambda: (0, 0)),
            pl.BlockSpec((T, 1), lambda: (0, 0)),
            pl.BlockSpec((T, 1), lambda: (0, 0)),
            pl.BlockSpec((MAX_TILES, 1), lambda: (0, 0)),
            pl.BlockSpec((MAX_TILES, 1), lambda: (0, 0)),
            pl.BlockSpec((MAX_SUPER, 1), lambda: (0, 0)),
        ],
        out_shape=[
            jax.ShapeDtypeStruct((T, 1), jnp.int32),
            jax.ShapeDtypeStruct((T, 1), jnp.int32),
            jax.ShapeDtypeStruct((T, 1), jnp.float32),
            jax.ShapeDtypeStruct((T, 1), jnp.float32),
            jax.ShapeDtypeStruct((MAX_TILES, 1), jnp.int32),
            jax.ShapeDtypeStruct((MAX_TILES, 1), jnp.int32),
            jax.ShapeDtypeStruct((MAX_SUPER, 1), jnp.int32),
        ],
    )(logits)
    pa, pb, ga, gb, te, lv, ls = outs
    return (pa.reshape(T), pb.reshape(T), ga.reshape(T), gb.reshape(T),
            te.reshape(MAX_TILES), lv.reshape(MAX_TILES),
            ls.reshape(MAX_SUPER))


def _sc_mesh():
    return plsc.VectorSubcoreMesh(
        core_axis_name="c", subcore_axis_name="s",
        num_cores=NC, num_subcores=NS)


def _dispatch_body(x_hbm, pa_hbm, pb_hbm, xs_hbm, xbuf, ia, ib, sem):
    wid = lax.axis_index("s") * NC + lax.axis_index("c")
    base = wid * TPW

    def step(c, carry):
        tb = base + c * CH
        pltpu.sync_copy(x_hbm.at[pl.ds(tb, CH)], xbuf)
        pltpu.sync_copy(pa_hbm.at[pl.ds(tb, CH)], ia)
        pltpu.sync_copy(pb_hbm.at[pl.ds(tb, CH)], ib)
        cpa = pltpu.async_copy(xbuf, xs_hbm.at[ia], sem)
        cpb = pltpu.async_copy(xbuf, xs_hbm.at[ib], sem)
        cpa.wait()
        cpb.wait()
        return carry

    lax.fori_loop(0, NCHUNK, step, 0)


def _dispatch_call(x, pa, pb):
    fn = functools.partial(
        pl.kernel,
        out_type=jax.ShapeDtypeStruct((N_PAD, D), jnp.float32),
        mesh=_sc_mesh(),
        scratch_types=[
            pltpu.VMEM((CH, D), jnp.float32),
            pltpu.VMEM((CH,), jnp.int32),
            pltpu.VMEM((CH,), jnp.int32),
            pltpu.SemaphoreType.DMA,
        ],
    )(_dispatch_body)
    return fn(x, pa, pb)


def _ffn_body(te_ref, lv_ref, ls_ref, xs_ref, w1_ref, w2_ref, out_ref):
    f = pl.program_id(1)
    ts = pl.program_id(2)
    t = pl.program_id(0) * TSUB + ts

    @pl.when(lv_ref[t] > 0)
    def _():
        x = xs_ref[pl.ds(ts * TILE_M, TILE_M), :]
        w1 = w1_ref[0]
        h = lax.dot_general(x, w1, (((1,), (1,)), ((), ())),
                            precision=lax.Precision.DEFAULT,
                            preferred_element_type=jnp.float32)
        h = h * 0.5 * (1.0 + lax.erf(h * (1.0 / math.sqrt(2.0))))
        w2 = w2_ref[0]
        contrib = lax.dot_general(h, w2, (((1,), (1,)), ((), ())),
                                  precision=lax.Precision.DEFAULT,
                                  preferred_element_type=jnp.float32)

        @pl.when(f == 0)
        def _():
            out_ref[pl.ds(ts * TILE_M, TILE_M), :] = contrib

        @pl.when(f > 0)
        def _():
            out_ref[pl.ds(ts * TILE_M, TILE_M), :] = (
                out_ref[pl.ds(ts * TILE_M, TILE_M), :] + contrib)


def _ffn_call(tile_expert, live, live_super, xs, w_fc, w_proj):
    grid_spec = pltpu.PrefetchScalarGridSpec(
        num_scalar_prefetch=3,
        grid=(MAX_SUPER, F // TILE_F, TSUB),
        in_specs=[
            pl.BlockSpec((SUPER_M, D),
                         lambda s, f, ts, te, lv, ls:
                         (jnp.where(ls[s] > 0, s, 0), 0)),
            pl.BlockSpec((1, TILE_F, D),
                         lambda s, f, ts, te, lv, ls:
                         (te[s * TSUB + ts], f, 0)),
            pl.BlockSpec((1, D, TILE_F),
                         lambda s, f, ts, te, lv, ls:
                         (te[s * TSUB + ts], 0, f)),
        ],
        out_specs=pl.BlockSpec((SUPER_M, D),
                               lambda s, f, ts, te, lv, ls: (s, 0)),
    )
    return pl.pallas_call(
        _ffn_body,
        grid_spec=grid_spec,
        out_shape=jax.ShapeDtypeStruct((N_PAD, D), jnp.float32),
    )(tile_expert, live, live_super, xs, w_fc, w_proj)


def _combine_body(rows_hbm, pa_hbm, pb_hbm, ga_hbm, gb_hbm, y_hbm,
                  bufa, bufb, bufo, ia, ib, ga, gb, sem):
    wid = lax.axis_index("s") * NC + lax.axis_index("c")
    base = wid * TPW

    def step(c, carry):
        tb = base + c * CH
        pltpu.sync_copy(pa_hbm.at[pl.ds(tb, CH)], ia)
        pltpu.sync_copy(pb_hbm.at[pl.ds(tb, CH)], ib)
        pltpu.sync_copy(ga_hbm.at[pl.ds(tb, CH)], ga)
        pltpu.sync_copy(gb_hbm.at[pl.ds(tb, CH)], gb)
        cpa = pltpu.async_copy(rows_hbm.at[ia], bufa, sem)
        cpb = pltpu.async_copy(rows_hbm.at[ib], bufb, sem)
        cpa.wait()
        cpb.wait()
        gveca = ga[...]
        gvecb = gb[...]
        for i in range(CH):
            gva = gveca[i]
            gvb = gvecb[i]

            def lane(j, c2):
                sl = pl.ds(pl.multiple_of(j * LANES, LANES), LANES)
                bufo[i, sl] = gva * bufa[i, sl] + gvb * bufb[i, sl]
                return c2

            lax.fori_loop(0, D // LANES, lane, 0)
        pltpu.sync_copy(bufo, y_hbm.at[pl.ds(tb, CH)])
        return carry

    lax.fori_loop(0, NCHUNK, step, 0)


def _combine_call(rows, pa, pb, ga, gb):
    fn = functools.partial(
        pl.kernel,
        out_type=jax.ShapeDtypeStruct((T, D), jnp.float32),
        mesh=_sc_mesh(),
        scratch_types=[
            pltpu.VMEM((CH, D), jnp.float32),
            pltpu.VMEM((CH, D), jnp.float32),
            pltpu.VMEM((CH, D), jnp.float32),
            pltpu.VMEM((CH,), jnp.int32),
            pltpu.VMEM((CH,), jnp.int32),
            pltpu.VMEM((CH,), jnp.float32),
            pltpu.VMEM((CH,), jnp.float32),
            pltpu.SemaphoreType.DMA,
        ],
    )(_combine_body)
    return fn(rows, pa, pb, ga, gb)


def kernel(hidden_states, w_gate, w_fc, w_proj):
    b, s, d = hidden_states.shape
    x = hidden_states.reshape(-1, d)
    logits = _logits_call(x, w_gate)
    pa, pb, ga, gb, tile_expert, live, live_super = _route_call(logits)
    xs = _dispatch_call(x, pa, pb)
    rows = _ffn_call(tile_expert, live, live_super, xs, w_fc, w_proj)
    y = _combine_call(rows, pa, pb, ga, gb)
    return y.reshape(b, s, d), logits


# speedup vs baseline: 1.5518x; 1.2229x over previous
import functools
import math

import jax
import jax.numpy as jnp
from jax import lax
from jax.experimental import pallas as pl
from jax.experimental.pallas import tpu as pltpu
from jax.experimental.pallas import tpu_sc as plsc

E = 8
TOP_K = 2
D = 2048
F = 8192
T = 4096
A = T * TOP_K

TILE_M = 256
TSUB = 2
SUPER_M = TILE_M * TSUB
MAX_SUPER = 24
MAX_TILES = MAX_SUPER * TSUB
N_PAD = MAX_SUPER * SUPER_M
TILE_F = 1024
TM_L = 512

NC = 2
NS = 16
NW = NC * NS
LANES = 16
TPW = T // NW
CH = 16
NCHUNK = TPW // CH


def _logits_body(x_ref, wg_ref, out_ref):
    out_ref[...] = lax.dot_general(
        x_ref[...], wg_ref[...], (((1,), (1,)), ((), ())),
        preferred_element_type=jnp.float32)


def _logits_call(x, w_gate):
    return pl.pallas_call(
        _logits_body,
        grid=(T // TM_L,),
        in_specs=[
            pl.BlockSpec((TM_L, D), lambda i: (i, 0)),
            pl.BlockSpec((E, D), lambda i: (0, 0)),
        ],
        out_specs=pl.BlockSpec((TM_L, E), lambda i: (i, 0)),
        out_shape=jax.ShapeDtypeStruct((T, E), jnp.float32),
    )(x, w_gate)


def _route_body(lg_ref, pa_ref, pb_ref, ga_ref, gb_ref, te_ref, lv_ref,
                ls_ref):
    lg = lg_ref[...]
    col = lax.broadcasted_iota(jnp.int32, (T, E), 1)
    m1 = jnp.max(lg, axis=1, keepdims=True)
    e1 = jnp.min(jnp.where(lg == m1, col, E), axis=1, keepdims=True)
    oh1 = col == e1
    lg2 = jnp.where(oh1, -jnp.inf, lg)
    m2 = jnp.max(lg2, axis=1, keepdims=True)
    e2 = jnp.min(jnp.where(lg2 == m2, col, E), axis=1, keepdims=True)
    oh2 = col == e2
    dexp = jnp.exp(m2 - m1)
    g1 = 1.0 / (1.0 + dexp)
    g2 = dexp * g1

    ohc = (oh1 | oh2).astype(jnp.float32)
    tri = lax.broadcasted_iota(jnp.int32, (512, 512), 0) > \
        lax.broadcasted_iota(jnp.int32, (512, 512), 1)
    trif = tri.astype(jnp.float32)
    run = jnp.zeros((1, E), jnp.float32)
    parts = []
    for r in range(T // 512):
        blk = lax.slice(ohc, (512 * r, 0), (512 * (r + 1), E))
        c_blk = lax.dot_general(trif, blk, (((1,), (0,)), ((), ())),
                                preferred_element_type=jnp.float32) + run
        parts.append(c_blk)
        run = run + jnp.sum(blk, axis=0, keepdims=True)
    cpre = jnp.concatenate(parts, axis=0)
    counts = run

    nsuper = jnp.floor((counts + (SUPER_M - 1)) / SUPER_M)
    pad_rows = nsuper * SUPER_M
    up = (lax.broadcasted_iota(jnp.int32, (E, E), 0) <
          lax.broadcasted_iota(jnp.int32, (E, E), 1)).astype(jnp.float32)
    pad_off = lax.dot_general(pad_rows, up, (((1,), (0,)), ((), ())),
                              preferred_element_type=jnp.float32)

    oh1f = oh1.astype(jnp.float32)
    oh2f = oh2.astype(jnp.float32)
    rank1 = jnp.sum(cpre * oh1f, axis=1, keepdims=True)
    rank2 = jnp.sum(cpre * oh2f, axis=1, keepdims=True)
    off1 = jnp.sum(pad_off * oh1f, axis=1, keepdims=True)
    off2 = jnp.sum(pad_off * oh2f, axis=1, keepdims=True)
    pa_ref[...] = (off1 + rank1).astype(jnp.int32)
    pb_ref[...] = (off2 + rank2).astype(jnp.int32)
    ga_ref[...] = g1
    gb_ref[...] = g2

    pad_off_i = pad_off.astype(jnp.int32)
    counts_i = counts.astype(jnp.int32)
    seg_end_i = (pad_off + pad_rows).astype(jnp.int32)
    ti = lax.broadcasted_iota(jnp.int32, (MAX_TILES, E), 0) * TILE_M
    te = jnp.sum((ti >= seg_end_i).astype(jnp.int32), axis=1, keepdims=True)
    te_ref[...] = jnp.minimum(te, E - 1)
    lv = (ti >= pad_off_i) & (ti < pad_off_i + counts_i)
    lv_ref[...] = jnp.sum(lv.astype(jnp.int32), axis=1, keepdims=True)
    si = lax.broadcasted_iota(jnp.int32, (MAX_SUPER, E), 0) * SUPER_M
    ls = (si >= pad_off_i) & (si < pad_off_i + counts_i)
    ls_ref[...] = jnp.sum(ls.astype(jnp.int32), axis=1, keepdims=True)


def _route_call(logits):
    outs = pl.pallas_call(
        _route_body,
        in_specs=[pl.BlockSpec((T, E), lambda: (0, 0))],
        out_specs=[
            pl.BlockSpec((T, 1), lambda: (0, 0)),
            pl.BlockSpec((T, 1), lambda: (0, 0)),
            pl.BlockSpec((T, 1), lambda: (0, 0)),
            pl.BlockSpec((T, 1), lambda: (0, 0)),
            pl.BlockSpec((MAX_TILES, 1), lambda: (0, 0)),
            pl.BlockSpec((MAX_TILES, 1), lambda: (0, 0)),
            pl.BlockSpec((MAX_SUPER, 1), lambda: (0, 0)),
        ],
        out_shape=[
            jax.ShapeDtypeStruct((T, 1), jnp.int32),
            jax.ShapeDtypeStruct((T, 1), jnp.int32),
            jax.ShapeDtypeStruct((T, 1), jnp.float32),
            jax.ShapeDtypeStruct((T, 1), jnp.float32),
            jax.ShapeDtypeStruct((MAX_TILES, 1), jnp.int32),
            jax.ShapeDtypeStruct((MAX_TILES, 1), jnp.int32),
            jax.ShapeDtypeStruct((MAX_SUPER, 1), jnp.int32),
        ],
    )(logits)
    pa, pb, ga, gb, te, lv, ls = outs
    return (pa.reshape(T), pb.reshape(T), ga.reshape(T), gb.reshape(T),
            te.reshape(MAX_TILES), lv.reshape(MAX_TILES),
            ls.reshape(MAX_SUPER))


def _sc_mesh():
    return plsc.VectorSubcoreMesh(
        core_axis_name="c", subcore_axis_name="s",
        num_cores=NC, num_subcores=NS)


CHD = 16
NCHD = TPW // CHD
NBUF = 3


def _dispatch_body(x_hbm, pa_hbm, pb_hbm, xs_hbm, idxa, idxb, xbuf,
                   lsem, ssem):
    wid = lax.axis_index("s") * NC + lax.axis_index("c")
    base = wid * TPW
    pltpu.sync_copy(pa_hbm.at[wid], idxa)
    pltpu.sync_copy(pb_hbm.at[wid], idxb)

    def startload(c, b):
        pltpu.async_copy(x_hbm.at[pl.ds(base + c * CHD, CHD)], xbuf.at[b],
                         lsem[b])

    def waitload(b):
        pltpu.make_async_copy(x_hbm.at[pl.ds(0, CHD)], xbuf.at[b],
                              lsem[b]).wait()

    def startscat(c, b):
        pltpu.async_copy(xbuf.at[b], xs_hbm.at[idxa.at[c]], ssem[b])
        pltpu.async_copy(xbuf.at[b], xs_hbm.at[idxb.at[c]], ssem[b])

    def waitscat(b):
        pltpu.make_async_copy(xbuf.at[b], xs_hbm.at[idxa.at[0]],
                              ssem[b]).wait()
        pltpu.make_async_copy(xbuf.at[b], xs_hbm.at[idxb.at[0]],
                              ssem[b]).wait()

    startload(0, 0)
    startload(1, 1)
    for c in range(NCHD):
        nxt = c + 2
        if nxt < NCHD:
            bn = nxt % NBUF
            if nxt >= NBUF:
                waitscat(bn)
            startload(nxt, bn)
        b = c % NBUF
        waitload(b)
        startscat(c, b)
    for b in range(NBUF):
        waitscat(b)


def _dispatch_call(x, pa, pb):
    fn = functools.partial(
        pl.kernel,
        out_type=jax.ShapeDtypeStruct((N_PAD, D), jnp.float32),
        mesh=_sc_mesh(),
        scratch_types=[
            pltpu.VMEM((NCHD, CHD), jnp.int32),
            pltpu.VMEM((NCHD, CHD), jnp.int32),
            pltpu.VMEM((NBUF, CHD, D), jnp.float32),
            [pltpu.SemaphoreType.DMA] * NBUF,
            [pltpu.SemaphoreType.DMA] * NBUF,
        ],
    )(_dispatch_body)
    return fn(x, pa.reshape(NW, NCHD, CHD), pb.reshape(NW, NCHD, CHD))


def _ffn_body(te_ref, lv_ref, ls_ref, xs_ref, w1_ref, w2_ref, out_ref):
    f = pl.program_id(1)
    ts = pl.program_id(2)
    t = pl.program_id(0) * TSUB + ts

    @pl.when(lv_ref[t] > 0)
    def _():
        x = xs_ref[pl.ds(ts * TILE_M, TILE_M), :]
        w1 = w1_ref[0]
        h = lax.dot_general(x, w1, (((1,), (1,)), ((), ())),
                            precision=lax.Precision.DEFAULT,
                            preferred_element_type=jnp.float32)
        h = h * 0.5 * (1.0 + lax.erf(h * (1.0 / math.sqrt(2.0))))
        w2 = w2_ref[0]
        contrib = lax.dot_general(h, w2, (((1,), (1,)), ((), ())),
                                  precision=lax.Precision.DEFAULT,
                                  preferred_element_type=jnp.float32)

        @pl.when(f == 0)
        def _():
            out_ref[pl.ds(ts * TILE_M, TILE_M), :] = contrib

        @pl.when(f > 0)
        def _():
            out_ref[pl.ds(ts * TILE_M, TILE_M), :] = (
                out_ref[pl.ds(ts * TILE_M, TILE_M), :] + contrib)


def _ffn_call(tile_expert, live, live_super, xs, w_fc, w_proj):
    grid_spec = pltpu.PrefetchScalarGridSpec(
        num_scalar_prefetch=3,
        grid=(MAX_SUPER, F // TILE_F, TSUB),
        in_specs=[
            pl.BlockSpec((SUPER_M, D),
                         lambda s, f, ts, te, lv, ls:
                         (jnp.where(ls[s] > 0, s, 0), 0)),
            pl.BlockSpec((1, TILE_F, D),
                         lambda s, f, ts, te, lv, ls:
                         (te[s * TSUB + ts],
                          jnp.where(ls[s] > 0, f, F // TILE_F - 1), 0)),
            pl.BlockSpec((1, D, TILE_F),
                         lambda s, f, ts, te, lv, ls:
                         (te[s * TSUB + ts], 0,
                          jnp.where(ls[s] > 0, f, F // TILE_F - 1))),
        ],
        out_specs=pl.BlockSpec((SUPER_M, D),
                               lambda s, f, ts, te, lv, ls: (s, 0)),
    )
    return pl.pallas_call(
        _ffn_body,
        grid_spec=grid_spec,
        out_shape=jax.ShapeDtypeStruct((N_PAD, D), jnp.float32),
    )(tile_expert, live, live_super, xs, w_fc, w_proj)


CHC = 8
NCHC = TPW // CHC


def _combine_body(rows_hbm, pa_hbm, pb_hbm, ga_hbm, gb_hbm, y_hbm,
                  idxa, idxb, gabuf, gbbuf, bufa, bufb, bufo, gsem):
    wid = lax.axis_index("s") * NC + lax.axis_index("c")
    base = wid * TPW
    pltpu.sync_copy(pa_hbm.at[wid], idxa)
    pltpu.sync_copy(pb_hbm.at[wid], idxb)
    pltpu.sync_copy(ga_hbm.at[wid], gabuf.at[pl.ds(0, TPW)])
    pltpu.sync_copy(gb_hbm.at[wid], gbbuf.at[pl.ds(0, TPW)])

    def start(c, p):
        pltpu.async_copy(rows_hbm.at[idxa.at[c]], bufa.at[p], gsem[p])
        pltpu.async_copy(rows_hbm.at[idxb.at[c]], bufb.at[p], gsem[p])

    def wait(p):
        pltpu.make_async_copy(rows_hbm.at[idxa.at[0]], bufa.at[p],
                              gsem[p]).wait()
        pltpu.make_async_copy(rows_hbm.at[idxb.at[0]], bufb.at[p],
                              gsem[p]).wait()

    def compute(c, p):
        gveca = gabuf[pl.ds(c * CHC, LANES)]
        gvecb = gbbuf[pl.ds(c * CHC, LANES)]
        for i in range(CHC):
            gva = gveca[i]
            gvb = gvecb[i]

            def lane(j, c2):
                sl = pl.ds(pl.multiple_of(j * LANES, LANES), LANES)
                bufo[i, sl] = gva * bufa[p, i, sl] + gvb * bufb[p, i, sl]
                return c2

            lax.fori_loop(0, D // LANES, lane, 0, unroll=4)
        pltpu.sync_copy(bufo, y_hbm.at[pl.ds(base + c * CHC, CHC)])

    start(0, 0)
    start(1, 1)
    for c in range(NCHC):
        nxt = c + 2
        if nxt < NCHC:
            start(nxt, nxt % 3)
        wait(c % 3)
        compute(c, c % 3)


def _combine_call(rows, pa, pb, ga, gb):
    fn = functools.partial(
        pl.kernel,
        out_type=jax.ShapeDtypeStruct((T, D), jnp.float32),
        mesh=_sc_mesh(),
        scratch_types=[
            pltpu.VMEM((NCHC, CHC), jnp.int32),
            pltpu.VMEM((NCHC, CHC), jnp.int32),
            pltpu.VMEM((TPW + LANES,), jnp.float32),
            pltpu.VMEM((TPW + LANES,), jnp.float32),
            pltpu.VMEM((3, CHC, D), jnp.float32),
            pltpu.VMEM((3, CHC, D), jnp.float32),
            pltpu.VMEM((CHC, D), jnp.float32),
            [pltpu.SemaphoreType.DMA] * 3,
        ],
    )(_combine_body)
    return fn(rows, pa.reshape(NW, NCHC, CHC), pb.reshape(NW, NCHC, CHC),
              ga.reshape(NW, TPW), gb.reshape(NW, TPW))


def kernel(hidden_states, w_gate, w_fc, w_proj):
    b, s, d = hidden_states.shape
    x = hidden_states.reshape(-1, d)
    logits = _logits_call(x, w_gate)
    pa, pb, ga, gb, tile_expert, live, live_super = _route_call(logits)
    xs = _dispatch_call(x, pa, pb)
    rows = _ffn_call(tile_expert, live, live_super, xs, w_fc, w_proj)
    y = _combine_call(rows, pa, pb, ga, gb)
    return y.reshape(b, s, d), logits


# speedup vs baseline: 1.5663x; 1.0093x over previous
import functools
import math

import jax
import jax.numpy as jnp
from jax import lax
from jax.experimental import pallas as pl
from jax.experimental.pallas import tpu as pltpu
from jax.experimental.pallas import tpu_sc as plsc

E = 8
TOP_K = 2
D = 2048
F = 8192
T = 4096
A = T * TOP_K

TILE_M = 256
TSUB = 2
SUPER_M = TILE_M * TSUB
MAX_SUPER = 24
MAX_TILES = MAX_SUPER * TSUB
N_PAD = MAX_SUPER * SUPER_M
TILE_F = 1024
TM_L = 512

NC = 2
NS = 16
NW = NC * NS
LANES = 16
TPW = T // NW
CH = 16
NCHUNK = TPW // CH


def _logits_body(x_ref, wg_ref, out_ref):
    out_ref[...] = lax.dot_general(
        x_ref[...], wg_ref[...], (((1,), (1,)), ((), ())),
        preferred_element_type=jnp.float32)


def _logits_call(x, w_gate):
    return pl.pallas_call(
        _logits_body,
        grid=(T // TM_L,),
        in_specs=[
            pl.BlockSpec((TM_L, D), lambda i: (i, 0)),
            pl.BlockSpec((E, D), lambda i: (0, 0)),
        ],
        out_specs=pl.BlockSpec((TM_L, E), lambda i: (i, 0)),
        out_shape=jax.ShapeDtypeStruct((T, E), jnp.float32),
    )(x, w_gate)


def _route_body(lg_ref, pa_ref, pb_ref, ga_ref, gb_ref, te_ref, lv_ref,
                ls_ref):
    lg = lg_ref[...]
    col = lax.broadcasted_iota(jnp.int32, (T, E), 1)
    m1 = jnp.max(lg, axis=1, keepdims=True)
    e1 = jnp.min(jnp.where(lg == m1, col, E), axis=1, keepdims=True)
    oh1 = col == e1
    lg2 = jnp.where(oh1, -jnp.inf, lg)
    m2 = jnp.max(lg2, axis=1, keepdims=True)
    e2 = jnp.min(jnp.where(lg2 == m2, col, E), axis=1, keepdims=True)
    oh2 = col == e2
    dexp = jnp.exp(m2 - m1)
    g1 = 1.0 / (1.0 + dexp)
    g2 = dexp * g1

    ohc = (oh1 | oh2).astype(jnp.float32)
    tri = lax.broadcasted_iota(jnp.int32, (512, 512), 0) > \
        lax.broadcasted_iota(jnp.int32, (512, 512), 1)
    trif = tri.astype(jnp.float32)
    run = jnp.zeros((1, E), jnp.float32)
    parts = []
    for r in range(T // 512):
        blk = lax.slice(ohc, (512 * r, 0), (512 * (r + 1), E))
        c_blk = lax.dot_general(trif, blk, (((1,), (0,)), ((), ())),
                                preferred_element_type=jnp.float32) + run
        parts.append(c_blk)
        run = run + jnp.sum(blk, axis=0, keepdims=True)
    cpre = jnp.concatenate(parts, axis=0)
    counts = run

    nsuper = jnp.floor((counts + (SUPER_M - 1)) / SUPER_M)
    pad_rows = nsuper * SUPER_M
    up = (lax.broadcasted_iota(jnp.int32, (E, E), 0) <
          lax.broadcasted_iota(jnp.int32, (E, E), 1)).astype(jnp.float32)
    pad_off = lax.dot_general(pad_rows, up, (((1,), (0,)), ((), ())),
                              preferred_element_type=jnp.float32)

    oh1f = oh1.astype(jnp.float32)
    oh2f = oh2.astype(jnp.float32)
    rank1 = jnp.sum(cpre * oh1f, axis=1, keepdims=True)
    rank2 = jnp.sum(cpre * oh2f, axis=1, keepdims=True)
    off1 = jnp.sum(pad_off * oh1f, axis=1, keepdims=True)
    off2 = jnp.sum(pad_off * oh2f, axis=1, keepdims=True)
    pa_ref[...] = (off1 + rank1).astype(jnp.int32)
    pb_ref[...] = (off2 + rank2).astype(jnp.int32)
    ga_ref[...] = g1
    gb_ref[...] = g2

    pad_off_i = pad_off.astype(jnp.int32)
    counts_i = counts.astype(jnp.int32)
    seg_end_i = (pad_off + pad_rows).astype(jnp.int32)
    ti = lax.broadcasted_iota(jnp.int32, (MAX_TILES, E), 0) * TILE_M
    te = jnp.sum((ti >= seg_end_i).astype(jnp.int32), axis=1, keepdims=True)
    te_ref[...] = jnp.minimum(te, E - 1)
    lv = (ti >= pad_off_i) & (ti < pad_off_i + counts_i)
    lv_ref[...] = jnp.sum(lv.astype(jnp.int32), axis=1, keepdims=True)
    si = lax.broadcasted_iota(jnp.int32, (MAX_SUPER, E), 0) * SUPER_M
    ls = (si >= pad_off_i) & (si < pad_off_i + counts_i)
    ls_ref[...] = jnp.sum(ls.astype(jnp.int32), axis=1, keepdims=True)


def _route_call(logits):
    outs = pl.pallas_call(
        _route_body,
        in_specs=[pl.BlockSpec((T, E), lambda: (0, 0))],
        out_specs=[
            pl.BlockSpec((T, 1), lambda: (0, 0)),
            pl.BlockSpec((T, 1), lambda: (0, 0)),
            pl.BlockSpec((T, 1), lambda: (0, 0)),
            pl.BlockSpec((T, 1), lambda: (0, 0)),
            pl.BlockSpec((MAX_TILES, 1), lambda: (0, 0)),
            pl.BlockSpec((MAX_TILES, 1), lambda: (0, 0)),
            pl.BlockSpec((MAX_SUPER, 1), lambda: (0, 0)),
        ],
        out_shape=[
            jax.ShapeDtypeStruct((T, 1), jnp.int32),
            jax.ShapeDtypeStruct((T, 1), jnp.int32),
            jax.ShapeDtypeStruct((T, 1), jnp.float32),
            jax.ShapeDtypeStruct((T, 1), jnp.float32),
            jax.ShapeDtypeStruct((MAX_TILES, 1), jnp.int32),
            jax.ShapeDtypeStruct((MAX_TILES, 1), jnp.int32),
            jax.ShapeDtypeStruct((MAX_SUPER, 1), jnp.int32),
        ],
    )(logits)
    pa, pb, ga, gb, te, lv, ls = outs
    return (pa.reshape(T), pb.reshape(T), ga.reshape(T), gb.reshape(T),
            te.reshape(MAX_TILES), lv.reshape(MAX_TILES),
            ls.reshape(MAX_SUPER))


def _sc_mesh():
    return plsc.VectorSubcoreMesh(
        core_axis_name="c", subcore_axis_name="s",
        num_cores=NC, num_subcores=NS)


CHD = 16
NCHD = TPW // CHD
NBUF = 3


def _dispatch_body(x_hbm, pa_hbm, pb_hbm, xs_hbm, idxa, idxb, xbuf,
                   lsem, ssem):
    wid = lax.axis_index("s") * NC + lax.axis_index("c")
    base = wid * TPW
    pltpu.sync_copy(pa_hbm.at[wid], idxa)
    pltpu.sync_copy(pb_hbm.at[wid], idxb)

    def startload(c, b):
        pltpu.async_copy(x_hbm.at[pl.ds(base + c * CHD, CHD)], xbuf.at[b],
                         lsem[b])

    def waitload(b):
        pltpu.make_async_copy(x_hbm.at[pl.ds(0, CHD)], xbuf.at[b],
                              lsem[b]).wait()

    def startscat(c, b):
        pltpu.async_copy(xbuf.at[b], xs_hbm.at[idxa.at[c]], ssem[b])
        pltpu.async_copy(xbuf.at[b], xs_hbm.at[idxb.at[c]], ssem[b])

    def waitscat(b):
        pltpu.make_async_copy(xbuf.at[b], xs_hbm.at[idxa.at[0]],
                              ssem[b]).wait()
        pltpu.make_async_copy(xbuf.at[b], xs_hbm.at[idxb.at[0]],
                              ssem[b]).wait()

    startload(0, 0)
    startload(1, 1)
    for c in range(NCHD):
        nxt = c + 2
        if nxt < NCHD:
            bn = nxt % NBUF
            if nxt >= NBUF:
                waitscat(bn)
            startload(nxt, bn)
        b = c % NBUF
        waitload(b)
        startscat(c, b)
    for b in range(NBUF):
        waitscat(b)


def _dispatch_call(x, pa, pb):
    fn = functools.partial(
        pl.kernel,
        out_type=jax.ShapeDtypeStruct((N_PAD, D), jnp.float32),
        mesh=_sc_mesh(),
        scratch_types=[
            pltpu.VMEM((NCHD, CHD), jnp.int32),
            pltpu.VMEM((NCHD, CHD), jnp.int32),
            pltpu.VMEM((NBUF, CHD, D), jnp.float32),
            [pltpu.SemaphoreType.DMA] * NBUF,
            [pltpu.SemaphoreType.DMA] * NBUF,
        ],
    )(_dispatch_body)
    return fn(x, pa.reshape(NW, NCHD, CHD), pb.reshape(NW, NCHD, CHD))


def _ffn_body(te_ref, lv_ref, ls_ref, xs_ref, w1_ref, w2_ref, out_ref):
    f = pl.program_id(1)
    ts = pl.program_id(2)
    t = pl.program_id(0) * TSUB + ts

    @pl.when(lv_ref[t] > 0)
    def _():
        x = xs_ref[pl.ds(ts * TILE_M, TILE_M), :]
        w1 = w1_ref[0]
        h = lax.dot_general(x, w1, (((1,), (1,)), ((), ())),
                            precision=lax.Precision.DEFAULT,
                            preferred_element_type=jnp.float32)
        h = h * 0.5 * (1.0 + lax.erf(h * (1.0 / math.sqrt(2.0))))
        w2 = w2_ref[0]
        contrib = lax.dot_general(h, w2, (((1,), (1,)), ((), ())),
                                  precision=lax.Precision.DEFAULT,
                                  preferred_element_type=jnp.float32)

        @pl.when(f == 0)
        def _():
            out_ref[pl.ds(ts * TILE_M, TILE_M), :] = contrib

        @pl.when(f > 0)
        def _():
            out_ref[pl.ds(ts * TILE_M, TILE_M), :] = (
                out_ref[pl.ds(ts * TILE_M, TILE_M), :] + contrib)


def _ffn_call(tile_expert, live, live_super, xs, w_fc, w_proj):
    grid_spec = pltpu.PrefetchScalarGridSpec(
        num_scalar_prefetch=3,
        grid=(MAX_SUPER, F // TILE_F, TSUB),
        in_specs=[
            pl.BlockSpec((SUPER_M, D),
                         lambda s, f, ts, te, lv, ls:
                         (jnp.where(ls[s] > 0, s, 0), 0)),
            pl.BlockSpec((1, TILE_F, D),
                         lambda s, f, ts, te, lv, ls:
                         (te[s * TSUB + ts],
                          jnp.where(ls[s] > 0, f, F // TILE_F - 1), 0)),
            pl.BlockSpec((1, D, TILE_F),
                         lambda s, f, ts, te, lv, ls:
                         (te[s * TSUB + ts], 0,
                          jnp.where(ls[s] > 0, f, F // TILE_F - 1))),
        ],
        out_specs=pl.BlockSpec((SUPER_M, D),
                               lambda s, f, ts, te, lv, ls: (s, 0)),
    )
    return pl.pallas_call(
        _ffn_body,
        grid_spec=grid_spec,
        out_shape=jax.ShapeDtypeStruct((N_PAD, D), jnp.float32),
    )(tile_expert, live, live_super, xs, w_fc, w_proj)


CHC = 8
NCHC = TPW // CHC


def _combine_body(rows_hbm, pa_hbm, pb_hbm, ga_hbm, gb_hbm, y_hbm,
                  idxa, idxb, gabuf, gbbuf, bufa, bufb, bufo, gsem):
    wid = lax.axis_index("s") * NC + lax.axis_index("c")
    base = wid * TPW
    pltpu.sync_copy(pa_hbm.at[wid], idxa)
    pltpu.sync_copy(pb_hbm.at[wid], idxb)
    pltpu.sync_copy(ga_hbm.at[wid], gabuf.at[pl.ds(0, TPW)])
    pltpu.sync_copy(gb_hbm.at[wid], gbbuf.at[pl.ds(0, TPW)])

    def start(c, p):
        pltpu.async_copy(rows_hbm.at[idxa.at[c]], bufa.at[p], gsem[p])
        pltpu.async_copy(rows_hbm.at[idxb.at[c]], bufb.at[p], gsem[p])

    def wait(p):
        pltpu.make_async_copy(rows_hbm.at[idxa.at[0]], bufa.at[p],
                              gsem[p]).wait()
        pltpu.make_async_copy(rows_hbm.at[idxb.at[0]], bufb.at[p],
                              gsem[p]).wait()

    def compute(c, p):
        gveca = gabuf[pl.ds(c * CHC, LANES)]
        gvecb = gbbuf[pl.ds(c * CHC, LANES)]
        for i in range(CHC):
            gva = gveca[i]
            gvb = gvecb[i]

            def lane(j, c2):
                sl = pl.ds(pl.multiple_of(j * LANES, LANES), LANES)
                bufo[i, sl] = gva * bufa[p, i, sl] + gvb * bufb[p, i, sl]
                return c2

            lax.fori_loop(0, D // LANES, lane, 0, unroll=4)
        pltpu.sync_copy(bufo, y_hbm.at[pl.ds(base + c * CHC, CHC)])

    start(0, 0)

    def step(c2, carry):
        c0 = 2 * c2
        start(c0 + 1, 1)
        wait(0)
        compute(c0, 0)

        @pl.when(c2 < NCHC // 2 - 1)
        def _():
            start(c0 + 2, 0)

        wait(1)
        compute(c0 + 1, 1)
        return carry

    lax.fori_loop(0, NCHC // 2, step, 0)


def _combine_call(rows, pa, pb, ga, gb):
    fn = functools.partial(
        pl.kernel,
        out_type=jax.ShapeDtypeStruct((T, D), jnp.float32),
        mesh=_sc_mesh(),
        scratch_types=[
            pltpu.VMEM((NCHC, CHC), jnp.int32),
            pltpu.VMEM((NCHC, CHC), jnp.int32),
            pltpu.VMEM((TPW + LANES,), jnp.float32),
            pltpu.VMEM((TPW + LANES,), jnp.float32),
            pltpu.VMEM((2, CHC, D), jnp.float32),
            pltpu.VMEM((2, CHC, D), jnp.float32),
            pltpu.VMEM((CHC, D), jnp.float32),
            [pltpu.SemaphoreType.DMA] * 2,
        ],
    )(_combine_body)
    return fn(rows, pa.reshape(NW, NCHC, CHC), pb.reshape(NW, NCHC, CHC),
              ga.reshape(NW, TPW), gb.reshape(NW, TPW))


def kernel(hidden_states, w_gate, w_fc, w_proj):
    b, s, d = hidden_states.shape
    x = hidden_states.reshape(-1, d)
    logits = _logits_call(x, w_gate)
    pa, pb, ga, gb, tile_expert, live, live_super = _route_call(logits)
    xs = _dispatch_call(x, pa, pb)
    rows = _ffn_call(tile_expert, live, live_super, xs, w_fc, w_proj)
    y = _combine_call(rows, pa, pb, ga, gb)
    return y.reshape(b, s, d), logits
